# baseline (speedup 1.0000x reference)
import functools

import jax
import jax.numpy as jnp
from jax import lax
from jax.experimental import pallas as pl
from jax.experimental.pallas import tpu as pltpu
from jax.experimental.pallas import tpu_sc as plsc


def kernel(idx, mu_weight, sigma_weight):
    B = idx.shape[0]
    V, D = mu_weight.shape
    info = plsc.get_sparse_core_info()
    NC, NS, L = info.num_cores, info.num_subcores, info.num_lanes
    NW = NC * NS
    assert B % NW == 0 and D % L == 0
    bpw = B // NW

    mesh = plsc.VectorSubcoreMesh(core_axis_name="c", subcore_axis_name="s")

    @functools.partial(
        pl.kernel,
        mesh=mesh,
        compiler_params=pltpu.CompilerParams(use_tc_tiling_on_sc=False,
                                             skip_device_barrier=True),
        out_type=jax.ShapeDtypeStruct((2 * B, D), jnp.float32),
        scratch_types=[
            pltpu.VMEM((bpw,), jnp.int32),
            pltpu.VMEM((bpw,), jnp.int32),
            pltpu.VMEM((bpw,), jnp.int32),
            pltpu.VMEM((bpw, D), jnp.float32),
            pltpu.VMEM((bpw, D), jnp.float32),
            pltpu.SemaphoreType.DMA,
            pltpu.SemaphoreType.DMA,
            pltpu.SemaphoreType.DMA,
            pltpu.SemaphoreType.DMA,
        ],
    )
    def run(idx_hbm, mu_hbm, sig_hbm, out_hbm,
            idx_v, evn_v, odd_v, mu_v, sig_v,
            sem_mu, sem_sig, sem_omu, sem_osig):
        wid = lax.axis_index("s") * NC + lax.axis_index("c")
        base = wid * bpw
        pltpu.sync_copy(idx_hbm.at[pl.ds(base, bpw)], idx_v)
        mu_cp = pltpu.async_copy(mu_hbm.at[idx_v], mu_v, sem_mu)
        sig_cp = pltpu.async_copy(sig_hbm.at[idx_v], sig_v, sem_sig)

        lane = lax.iota(jnp.int32, L)
        for j in range(bpw // L):
            evn = (base + j * L + lane) * 2
            evn_v[pl.ds(j * L, L)] = evn
            odd_v[pl.ds(j * L, L)] = evn + 1

        mu_cp.wait()
        omu_cp = pltpu.async_copy(mu_v, out_hbm.at[evn_v], sem_omu)

        sig_cp.wait()
        rows_per_iter = 4

        def body(i, carry):
            r0 = i * rows_per_iter
            for rr in range(rows_per_iter):
                for j in range(D // L):
                    x = sig_v[r0 + rr, pl.ds(j * L, L)]
                    sig_v[r0 + rr, pl.ds(j * L, L)] = (
                        jnp.maximum(x, 0.0) + jnp.exp(jnp.minimum(x, 0.0)))
            return carry

        lax.fori_loop(0, bpw // rows_per_iter, body, 0)

        osig_cp = pltpu.async_copy(sig_v, out_hbm.at[odd_v], sem_osig)
        omu_cp.wait()
        osig_cp.wait()

    out2 = run(idx, mu_weight, sigma_weight)
    return out2.reshape(B, 2 * D)


# speedup vs baseline: 1.0031x; 1.0031x over previous
import functools

import jax
import jax.numpy as jnp
from jax import lax
from jax.experimental import pallas as pl
from jax.experimental.pallas import tpu as pltpu
from jax.experimental.pallas import tpu_sc as plsc


def kernel(idx, mu_weight, sigma_weight):
    B = idx.shape[0]
    V, D = mu_weight.shape
    info = plsc.get_sparse_core_info()
    NC, NS, L = info.num_cores, info.num_subcores, info.num_lanes
    NW = NC * NS
    assert B % NW == 0 and D % L == 0
    bpw = B // NW

    mesh = plsc.VectorSubcoreMesh(core_axis_name="c", subcore_axis_name="s")

    @functools.partial(
        pl.kernel,
        mesh=mesh,
        compiler_params=pltpu.CompilerParams(use_tc_tiling_on_sc=False),
        out_type=jax.ShapeDtypeStruct((2 * B, D), jnp.float32),
        scratch_types=[
            pltpu.VMEM((bpw,), jnp.int32),
            pltpu.VMEM((bpw,), jnp.int32),
            pltpu.VMEM((bpw,), jnp.int32),
            pltpu.VMEM((bpw, D), jnp.float32),
            pltpu.VMEM((bpw, D), jnp.float32),
            pltpu.SemaphoreType.DMA,
            pltpu.SemaphoreType.DMA,
            pltpu.SemaphoreType.DMA,
            pltpu.SemaphoreType.DMA,
        ],
    )
    def run(idx_hbm, mu_hbm, sig_hbm, out_hbm,
            idx_v, evn_v, odd_v, mu_v, sig_v,
            sem_mu, sem_sig, sem_omu, sem_osig):
        wid = lax.axis_index("s") * NC + lax.axis_index("c")
        base = wid * bpw
        pltpu.sync_copy(idx_hbm.at[pl.ds(base, bpw)], idx_v)
        mu_cp = pltpu.async_copy(mu_hbm.at[idx_v], mu_v, sem_mu)
        sig_cp = pltpu.async_copy(sig_hbm.at[idx_v], sig_v, sem_sig)

        lane = lax.iota(jnp.int32, L)
        for j in range(bpw // L):
            evn = (base + j * L + lane) * 2
            evn_v[pl.ds(j * L, L)] = evn
            odd_v[pl.ds(j * L, L)] = evn + 1

        mu_cp.wait()
        omu_cp = pltpu.async_copy(mu_v, out_hbm.at[evn_v], sem_omu)

        sig_cp.wait()
        rows_per_iter = 4

        def body(i, carry):
            r0 = i * rows_per_iter
            for rr in range(rows_per_iter):
                for j in range(D // L):
                    x = sig_v[r0 + rr, pl.ds(j * L, L)]
                    sig_v[r0 + rr, pl.ds(j * L, L)] = (
                        jnp.maximum(x, 0.0) + jnp.exp(jnp.minimum(x, 0.0)))
            return carry

        lax.fori_loop(0, bpw // rows_per_iter, body, 0)

        osig_cp = pltpu.async_copy(sig_v, out_hbm.at[odd_v], sem_osig)
        omu_cp.wait()
        osig_cp.wait()

    out2 = run(idx, mu_weight, sigma_weight)
    return out2.reshape(B, 2 * D)


# speedup vs baseline: 1.1999x; 1.1962x over previous
import functools

import jax
import jax.numpy as jnp
from jax import lax
from jax.experimental import pallas as pl
from jax.experimental.pallas import tpu as pltpu
from jax.experimental.pallas import tpu_sc as plsc


def _slab_gather_kernel(B, V, D, NC, L, bpw, act):
    qtr = bpw // 4
    mesh = plsc.VectorSubcoreMesh(core_axis_name="c", subcore_axis_name="s")

    @functools.partial(
        pl.kernel,
        mesh=mesh,
        compiler_params=pltpu.CompilerParams(use_tc_tiling_on_sc=True),
        out_type=jax.ShapeDtypeStruct((B, D), jnp.float32),
        scratch_types=[
            pltpu.VMEM((bpw,), jnp.int32),
            pltpu.VMEM((2, qtr, 8, D), jnp.float32),
            pltpu.VMEM((bpw, D), jnp.float32),
            pltpu.SemaphoreType.DMA,
            pltpu.SemaphoreType.DMA,
        ],
    )
    def run(idx_hbm, tbl_hbm, out_hbm, idx_v, slab_v, row_v, sem0, sem1):
        wid = lax.axis_index("s") * NC + lax.axis_index("c")
        base = wid * bpw
        pltpu.sync_copy(idx_hbm.at[pl.ds(base, bpw)], idx_v)
        sems = (sem0, sem1)

        def issue(p):
            cps = []
            for i in range(qtr // L):
                rv = idx_v[pl.ds(p * qtr + i * L, L)]
                for l in range(L):
                    jj = i * L + l
                    t8 = pl.multiple_of((rv[l] >> 3) * 8, 8)
                    cps.append(pltpu.async_copy(
                        tbl_hbm.at[pl.ds(t8, 8), :],
                        slab_v.at[p % 2, jj], sems[p % 2]))
            return cps

        pend = {0: issue(0)}
        for p in range(4):
            if p + 1 < 4:
                pend[p + 1] = issue(p + 1)
            for cp in pend.pop(p):
                cp.wait()
            for i in range(qtr // L):
                rv = idx_v[pl.ds(p * qtr + i * L, L)]
                for l in range(L):
                    jj = i * L + l
                    j = p * qtr + jj
                    q = rv[l] & 7
                    for cb in range(D // L):
                        v = slab_v[p % 2, jj, q, pl.ds(cb * L, L)]
                        if act:
                            v = (jnp.maximum(v, 0.0)
                                 + jnp.exp(jnp.minimum(v, 0.0)))
                        row_v[j, pl.ds(cb * L, L)] = v

        pltpu.sync_copy(row_v, out_hbm.at[pl.ds(base, bpw)])

    return run


def kernel(idx, mu_weight, sigma_weight):
    B = idx.shape[0]
    V, D = mu_weight.shape
    info = plsc.get_sparse_core_info()
    NC, NS, L = info.num_cores, info.num_subcores, info.num_lanes
    NW = NC * NS
    assert B % (L * NW) == 0 and D % L == 0 and V % 8 == 0
    bpw = B // NW

    mu_run = _slab_gather_kernel(B, V, D, NC, L, bpw, act=False)
    sig_run = _slab_gather_kernel(B, V, D, NC, L, bpw, act=True)
    mu_emb = mu_run(idx, mu_weight)
    sig_act = sig_run(idx, sigma_weight)
    return jnp.concatenate([mu_emb, sig_act], axis=1)
